# baseline (speedup 1.0000x reference)
import functools

import jax
import jax.numpy as jnp
from jax import lax
from jax.experimental import pallas as pl
from jax.experimental.pallas import tpu as pltpu
from jax.experimental.pallas import tpu_sc as plsc

NUM_CORES = 2
NUM_SUBCORES = 16
NUM_WORKERS = NUM_CORES * NUM_SUBCORES
LANES = 16

ROWS = 4 * 8192
D = 1024
ROWS_PER_WORKER = ROWS // NUM_WORKERS
CHUNK = 16
NCHUNKS = ROWS_PER_WORKER // CHUNK


def _body(x_hbm, keep_hbm, out_hbm, keep_v, buf_v):
    wid = lax.axis_index("s") * NUM_CORES + lax.axis_index("c")
    base = wid * ROWS_PER_WORKER

    pltpu.sync_copy(
        keep_hbm.at[pl.ds(base * LANES, ROWS_PER_WORKER * LANES)], keep_v)

    def chunk_body(g, _):
        row0 = base + g * CHUNK
        pltpu.sync_copy(x_hbm.at[pl.ds(row0, CHUNK)], buf_v)

        def row_body(r, _):
            kv = keep_v[pl.ds((g * CHUNK + r) * LANES, LANES)]

            def col_body(j, _):
                v = buf_v[r, pl.ds(j * LANES, LANES)]
                buf_v[r, pl.ds(j * LANES, LANES)] = v * kv
                return 0

            return lax.fori_loop(0, D // LANES, col_body, 0, unroll=8)

        lax.fori_loop(0, CHUNK, row_body, 0)
        pltpu.sync_copy(buf_v, out_hbm.at[pl.ds(row0, CHUNK)])
        return 0

    lax.fori_loop(0, NCHUNKS, chunk_body, 0)


@jax.jit
def _masked_zero(x2d, keep16):
    mesh = plsc.VectorSubcoreMesh(
        core_axis_name="c", subcore_axis_name="s",
        num_cores=NUM_CORES, num_subcores=NUM_SUBCORES)
    return pl.kernel(
        _body,
        out_type=jax.ShapeDtypeStruct((ROWS, D), jnp.float32),
        mesh=mesh,
        scratch_types=[
            pltpu.VMEM((ROWS_PER_WORKER * LANES,), jnp.float32),
            pltpu.VMEM((CHUNK, D), jnp.float32),
        ],
    )(x2d, keep16)


def kernel(x, x_pad_mask):
    x2d = x.reshape(ROWS, D)
    keep = jnp.where(x_pad_mask.reshape(ROWS), 0.0, 1.0).astype(jnp.float32)
    keep16 = jnp.broadcast_to(keep[:, None], (ROWS, LANES)).reshape(ROWS * LANES)
    out = _masked_zero(x2d, keep16)
    return out.reshape(x.shape)


# speedup vs baseline: 1.6390x; 1.6390x over previous
import jax
import jax.numpy as jnp
from jax import lax
from jax.experimental import pallas as pl
from jax.experimental.pallas import tpu as pltpu
from jax.experimental.pallas import tpu_sc as plsc

NUM_CORES = 2
NUM_SUBCORES = 16
NUM_WORKERS = NUM_CORES * NUM_SUBCORES
LANES = 16

ROWS = 4 * 8192
D = 1024
ROWS_PER_WORKER = ROWS // NUM_WORKERS
CHUNK = 16
NCHUNKS = ROWS_PER_WORKER // CHUNK
NBUF = 4


def _body(x_hbm, keep_hbm, out_hbm, keep_v, b0, b1, b2, b3, in_sems, out_sems):
    bufs = [b0, b1, b2, b3]
    wid = lax.axis_index("s") * NUM_CORES + lax.axis_index("c")
    base = wid * ROWS_PER_WORKER

    pltpu.sync_copy(
        keep_hbm.at[pl.ds(base * LANES, ROWS_PER_WORKER * LANES)], keep_v)

    def start_in(g, slot):
        pltpu.make_async_copy(
            x_hbm.at[pl.ds(base + g * CHUNK, CHUNK)],
            bufs[slot], in_sems.at[slot]).start()

    def start_out(g, slot):
        pltpu.make_async_copy(
            bufs[slot],
            out_hbm.at[pl.ds(base + g * CHUNK, CHUNK)],
            out_sems.at[slot]).start()

    def wait_in(g, slot):
        pltpu.make_async_copy(
            x_hbm.at[pl.ds(base + g * CHUNK, CHUNK)],
            bufs[slot], in_sems.at[slot]).wait()

    def wait_out(g, slot):
        pltpu.make_async_copy(
            bufs[slot],
            out_hbm.at[pl.ds(base + g * CHUNK, CHUNK)],
            out_sems.at[slot]).wait()

    def compute(g, slot):
        buf = bufs[slot]

        def row_body(r, _):
            kv = keep_v[pl.ds((g * CHUNK + r) * LANES, LANES)]

            def col_body(j, _):
                v = buf[r, pl.ds(j * LANES, LANES)]
                buf[r, pl.ds(j * LANES, LANES)] = v * kv
                return 0

            return lax.fori_loop(0, D // LANES, col_body, 0, unroll=8)

        lax.fori_loop(0, CHUNK, row_body, 0)

    start_in(0, 0)
    start_in(1, 1)

    def group_body(go, _):
        for i in range(NBUF):
            g = go * NBUF + i
            gp = g + 2
            slot_p = (i + 2) % NBUF

            @pl.when(gp < NCHUNKS)
            def _():
                @pl.when(gp >= NBUF)
                def _():
                    wait_out(gp - NBUF, slot_p)
                start_in(gp, slot_p)

            wait_in(g, i)
            compute(g, i)
            start_out(g, i)
        return 0

    lax.fori_loop(0, NCHUNKS // NBUF, group_body, 0)

    wait_out(NCHUNKS - 2, (NCHUNKS - 2) % NBUF)
    wait_out(NCHUNKS - 1, (NCHUNKS - 1) % NBUF)


@jax.jit
def _masked_zero(x2d, keep16):
    mesh = plsc.VectorSubcoreMesh(
        core_axis_name="c", subcore_axis_name="s",
        num_cores=NUM_CORES, num_subcores=NUM_SUBCORES)
    return pl.kernel(
        _body,
        out_type=jax.ShapeDtypeStruct((ROWS, D), jnp.float32),
        mesh=mesh,
        scratch_types=[
            pltpu.VMEM((ROWS_PER_WORKER * LANES,), jnp.float32),
        ] + [pltpu.VMEM((CHUNK, D), jnp.float32) for _ in range(NBUF)] + [
            pltpu.SemaphoreType.DMA((NBUF,)),
            pltpu.SemaphoreType.DMA((NBUF,)),
        ],
    )(x2d, keep16)


def kernel(x, x_pad_mask):
    x2d = x.reshape(ROWS, D)
    keep = jnp.where(x_pad_mask.reshape(ROWS), 0.0, 1.0).astype(jnp.float32)
    keep16 = jnp.broadcast_to(keep[:, None], (ROWS, LANES)).reshape(ROWS * LANES)
    out = _masked_zero(x2d, keep16)
    return out.reshape(x.shape)
